# baseline (speedup 1.0000x reference)
import functools

import jax
import jax.numpy as jnp
import numpy as np
from jax import lax
from jax.experimental import pallas as pl
from jax.experimental.pallas import tpu as pltpu
from jax.experimental.pallas import tpu_sc as plsc

_N = 64
_TRI = _N * (_N + 1) // 2
_ROWS = 1024 * 16
_NWORKERS = 32
_CHUNK = _ROWS // _NWORKERS
_M = 8
_BLOCKS = _CHUNK // _M
_NBUF = 2
_NCHUNKS = _TRI // 16

_r, _c = np.triu_indices(_N)
_TRIU_IDX = (_r * _N + _c).astype(np.int32)

_mesh = plsc.VectorSubcoreMesh(core_axis_name="c", subcore_axis_name="s")


@functools.partial(
    pl.kernel,
    out_type=jax.ShapeDtypeStruct((_ROWS, _TRI), jnp.float32),
    mesh=_mesh,
    scratch_types=[
        pltpu.VMEM((_TRI,), jnp.int32),
        pltpu.VMEM((_NBUF, _M, _N * _N), jnp.float32),
        pltpu.VMEM((_NBUF, _M, _TRI), jnp.float32),
        pltpu.SemaphoreType.DMA,
        pltpu.SemaphoreType.DMA,
    ],
    compiler_params=pltpu.CompilerParams(needs_layout_passes=False),
)
def _triu_gather(in_hbm, idx_hbm, out_hbm, idx_v, inb, outb, insem, outsem):
    wid = lax.axis_index("s") * 2 + lax.axis_index("c")
    base = wid * _CHUNK
    pltpu.sync_copy(idx_hbm, idx_v)

    for u in range(_NBUF):
        pltpu.async_copy(
            in_hbm.at[pl.ds(base + u * _M, _M)], inb.at[u], insem
        )

    def outer(bb, carry):
        for u in range(_NBUF):
            b = bb * _NBUF + u
            row0 = base + b * _M
            pltpu.make_async_copy(
                in_hbm.at[pl.ds(0, _M)], inb.at[u], insem
            ).wait()
            @pl.when(b >= _NBUF)
            def _():
                pltpu.make_async_copy(
                    outb.at[u], out_hbm.at[pl.ds(0, _M)], outsem
                ).wait()

            def compact(k, carry2):
                col_idx = idx_v[pl.ds(k * 16, 16)]
                for m in range(_M):
                    row_idx = jnp.full((16,), m, jnp.int32)
                    g = plsc.load_gather(inb.at[u], [row_idx, col_idx])
                    outb.at[u][m, pl.ds(k * 16, 16)] = g
                return carry2

            lax.fori_loop(0, _NCHUNKS, compact, 0, unroll=2)

            pltpu.async_copy(outb.at[u], out_hbm.at[pl.ds(row0, _M)], outsem)

            @pl.when(b + _NBUF < _BLOCKS)
            def _():
                pltpu.async_copy(
                    in_hbm.at[pl.ds(row0 + _NBUF * _M, _M)], inb.at[u], insem
                )

        return carry

    lax.fori_loop(0, _BLOCKS // _NBUF, outer, 0)

    for u in range(_NBUF):
        pltpu.make_async_copy(
            outb.at[u], out_hbm.at[pl.ds(0, _M)], outsem
        ).wait()


def kernel(input):
    x = input.reshape(_ROWS, _N * _N)
    out = _triu_gather(x, jnp.asarray(_TRIU_IDX))
    return out.reshape(input.shape[0], -1)


# speedup vs baseline: 1.1534x; 1.1534x over previous
import functools

import jax
import jax.numpy as jnp
from jax import lax
from jax.experimental import pallas as pl
from jax.experimental.pallas import tpu as pltpu
from jax.experimental.pallas import tpu_sc as plsc

_N = 64
_TRI = _N * (_N + 1) // 2
_ROWS = 1024 * 16
_NWORKERS = 32
_CHUNK = _ROWS // _NWORKERS
_M = 8
_BLOCKS = _CHUNK // _M
_NBUF = 2

_OFF = [0] * (_N + 1)
for _i in range(_N):
    _OFF[_i + 1] = _OFF[_i] + (_N - _i)

_GATHER_LO = _TRI - 128
_WINDOWS = []
for _i in range(_N - 15):
    _s, _d, _w = _i * (_N + 1), _OFF[_i], _N - _i
    for _j in range(_w // 16):
        _WINDOWS.append((_s + 16 * _j, _d + 16 * _j))
    if _w % 16:
        _WINDOWS.append((_s + _w - 16, _d + _w - 16))

import numpy as _np

_rr, _cc = _np.triu_indices(_N)
_TAIL_IDX = (_rr * _N + _cc)[_GATHER_LO:].astype(_np.int32)

_mesh = plsc.VectorSubcoreMesh(core_axis_name="c", subcore_axis_name="s")


@functools.partial(
    pl.kernel,
    out_type=jax.ShapeDtypeStruct((_ROWS, _TRI), jnp.float32),
    mesh=_mesh,
    scratch_types=[
        pltpu.VMEM((128,), jnp.int32),
        pltpu.VMEM((_NBUF, _M, _N * _N), jnp.float32),
        pltpu.VMEM((_NBUF, _M, _TRI), jnp.float32),
        pltpu.SemaphoreType.DMA,
        pltpu.SemaphoreType.DMA,
    ],
    compiler_params=pltpu.CompilerParams(
        needs_layout_passes=False, use_tc_tiling_on_sc=False
    ),
)
def _triu_compact(in_hbm, idx_hbm, out_hbm, idx_v, inb, outb, insem, outsem):
    pltpu.sync_copy(idx_hbm, idx_v)
    wid = lax.axis_index("s") * 2 + lax.axis_index("c")
    base = wid * _CHUNK

    for u in range(_NBUF):
        pltpu.async_copy(in_hbm.at[pl.ds(base + u * _M, _M)], inb.at[u], insem)

    def outer(bb, carry):
        for u in range(_NBUF):
            b = bb * _NBUF + u
            row0 = base + b * _M
            pltpu.make_async_copy(
                in_hbm.at[pl.ds(0, _M)], inb.at[u], insem
            ).wait()

            @pl.when(b >= _NBUF)
            def _():
                pltpu.make_async_copy(
                    outb.at[u], out_hbm.at[pl.ds(0, _M)], outsem
                ).wait()

            for m in range(_M):
                for s, d in _WINDOWS:
                    outb.at[u][m, pl.ds(d, 16)] = inb.at[u][m, pl.ds(s, 16)]
                for t in range(8):
                    g = plsc.load_gather(
                        inb.at[u],
                        [jnp.full((16,), m, jnp.int32), idx_v[pl.ds(t * 16, 16)]],
                    )
                    outb.at[u][m, pl.ds(_GATHER_LO + t * 16, 16)] = g

            pltpu.async_copy(outb.at[u], out_hbm.at[pl.ds(row0, _M)], outsem)

            @pl.when(b + _NBUF < _BLOCKS)
            def _():
                pltpu.async_copy(
                    in_hbm.at[pl.ds(row0 + _NBUF * _M, _M)], inb.at[u], insem
                )

        return carry

    lax.fori_loop(0, _BLOCKS // _NBUF, outer, 0)

    for u in range(_NBUF):
        pltpu.make_async_copy(
            outb.at[u], out_hbm.at[pl.ds(0, _M)], outsem
        ).wait()


def kernel(input):
    x = input.reshape(_ROWS, _N * _N)
    out = _triu_compact(x, jnp.asarray(_TAIL_IDX))
    return out.reshape(input.shape[0], -1)


# speedup vs baseline: 1.1537x; 1.0002x over previous
import functools

import jax
import jax.numpy as jnp
import numpy as np
from jax import lax
from jax.experimental import pallas as pl
from jax.experimental.pallas import tpu as pltpu
from jax.experimental.pallas import tpu_sc as plsc

_N = 64
_MAT = _N * _N
_TRI = _N * (_N + 1) // 2
_ROWS = 1024 * 16
_NWORKERS = 32
_CHUNK = _ROWS // _NWORKERS
_M = 8
_BLOCKS = _CHUNK // _M
_NBUF = 2

_OFF = [0] * (_N + 1)
for _i in range(_N):
    _OFF[_i + 1] = _OFF[_i] + (_N - _i)

_GATHER_LO = _TRI - 128
_WINDOWS = []
for _i in range(_N - 15):
    _s, _d, _w = _i * (_N + 1), _OFF[_i], _N - _i
    for _j in range(_w // 16):
        _WINDOWS.append((_s + 16 * _j, _d + 16 * _j))
    if _w % 16:
        _WINDOWS.append((_s + _w - 16, _d + _w - 16))

_rr, _cc = np.triu_indices(_N)
_TAIL_IDX = (_rr * _N + _cc)[_GATHER_LO:].astype(np.int32)

_mesh = plsc.VectorSubcoreMesh(core_axis_name="c", subcore_axis_name="s")


@functools.partial(
    pl.kernel,
    out_type=jax.ShapeDtypeStruct((_ROWS * _TRI,), jnp.float32),
    mesh=_mesh,
    scratch_types=[
        pltpu.VMEM((128,), jnp.int32),
        pltpu.VMEM((_NBUF, _M * _MAT), jnp.float32),
        pltpu.VMEM((_NBUF, _M * _TRI), jnp.float32),
        pltpu.SemaphoreType.DMA,
        pltpu.SemaphoreType.DMA,
    ],
    compiler_params=pltpu.CompilerParams(
        needs_layout_passes=False, use_tc_tiling_on_sc=False
    ),
)
def _triu_compact(in_hbm, idx_hbm, out_hbm, idx_v, inb, outb, insem, outsem):
    pltpu.sync_copy(idx_hbm, idx_v)
    wid = lax.axis_index("s") * 2 + lax.axis_index("c")
    base = wid * _CHUNK

    for u in range(_NBUF):
        pltpu.async_copy(
            in_hbm.at[pl.ds((base + u * _M) * _MAT, _M * _MAT)],
            inb.at[u],
            insem,
        )

    def outer(bb, carry):
        for u in range(_NBUF):
            b = bb * _NBUF + u
            row0 = base + b * _M
            pltpu.make_async_copy(
                in_hbm.at[pl.ds(0, _M * _MAT)], inb.at[u], insem
            ).wait()

            @pl.when(b >= _NBUF)
            def _():
                pltpu.make_async_copy(
                    outb.at[u], out_hbm.at[pl.ds(0, _M * _TRI)], outsem
                ).wait()

            for m in range(_M):
                for s, d in _WINDOWS:
                    outb.at[u][pl.ds(m * _TRI + d, 16)] = inb.at[u][
                        pl.ds(m * _MAT + s, 16)
                    ]
                for t in range(8):
                    g = plsc.load_gather(
                        inb.at[u],
                        [idx_v[pl.ds(t * 16, 16)] + jnp.full((16,), m * _MAT, jnp.int32)],
                    )
                    outb.at[u][pl.ds(m * _TRI + _GATHER_LO + t * 16, 16)] = g

            pltpu.async_copy(
                outb.at[u], out_hbm.at[pl.ds(row0 * _TRI, _M * _TRI)], outsem
            )

            @pl.when(b + _NBUF < _BLOCKS)
            def _():
                pltpu.async_copy(
                    in_hbm.at[pl.ds((row0 + _NBUF * _M) * _MAT, _M * _MAT)],
                    inb.at[u],
                    insem,
                )

        return carry

    lax.fori_loop(0, _BLOCKS // _NBUF, outer, 0)

    for u in range(_NBUF):
        pltpu.make_async_copy(
            outb.at[u], out_hbm.at[pl.ds(0, _M * _TRI)], outsem
        ).wait()


def kernel(input):
    x = input.reshape(-1)
    out = _triu_compact(x, jnp.asarray(_TAIL_IDX))
    return out.reshape(input.shape[0], -1)


# speedup vs baseline: 1.2994x; 1.1263x over previous
import functools

import jax
import jax.numpy as jnp
import numpy as np
from jax import lax
from jax.experimental import pallas as pl
from jax.experimental.pallas import tpu as pltpu
from jax.experimental.pallas import tpu_sc as plsc

_N = 64
_TRI = _N * (_N + 1) // 2
_H = 16
_B = 1024
_NCOLS = _H * _TRI
_NJ = _NCOLS // 128
_NBB = _B // 128
_NBLK = _NJ * _NBB
_PER_W = _NBLK // 32
_MAXP = 16
_DSTRIDE = 24
_TROWS = 272

_r_, _c_ = np.triu_indices(_N)
_OFFS = np.concatenate([[0], np.cumsum(np.arange(_N, 0, -1))])
_cols = np.arange(_NCOLS)
_h = _cols // _TRI
_k = _cols % _TRI
_rm = np.searchsorted(_OFFS, _k, side="right") - 1
_cm = _k - _OFFS[_rm] + _rm
_FLAT = _h * (_N * _N) + _rm * _N + _cm

_DESC = np.zeros((_TROWS, _DSTRIDE), np.int32)
_CMAP = np.zeros((_TROWS, 128), np.int32)
for _j in range(_NJ):
    _f = _FLAT[_j * 128 : (_j + 1) * 128]
    _runs = []
    _s = _p = int(_f[0])
    for _v in _f[1:]:
        _v = int(_v)
        if _v != _p + 1:
            _runs.append((_s, _p))
            _s = _v
        _p = _v
    _runs.append((_s, _p))
    _pieces = []
    for _a, _b2 in _runs:
        _ca = _a - (_a % 8)
        while _ca <= _b2:
            _pieces.append(min(_ca, _N * _N * _H - 16))
            _ca += 16
    assert len(_pieces) <= _MAXP
    _DESC[_j, 0] = len(_pieces)
    _DESC[_j, 1 : 1 + len(_pieces)] = _pieces
    for _q, _fl in enumerate(_f):
        for _pi, _ps in enumerate(_pieces):
            if _ps <= _fl < _ps + 16:
                _CMAP[_j, _q] = _pi * 16 + (int(_fl) - _ps)
                break
        else:
            raise AssertionError("uncovered column")
_DESC = _DESC.reshape(-1)
_CMAP = _CMAP.reshape(-1)

_mesh = plsc.VectorSubcoreMesh(core_axis_name="c", subcore_axis_name="s")


@functools.partial(
    pl.kernel,
    out_type=jax.ShapeDtypeStruct((_B, _NCOLS), jnp.float32),
    mesh=_mesh,
    scratch_types=[
        pltpu.VMEM((10 * _DSTRIDE,), jnp.int32),
        pltpu.VMEM((10 * 128,), jnp.int32),
        pltpu.VMEM((_MAXP * 16, 128), jnp.float32),
        pltpu.VMEM((_MAXP * 16, 128), jnp.float32),
        pltpu.VMEM((128, 128), jnp.float32),
        pltpu.VMEM((128, 128), jnp.float32),
        pltpu.SemaphoreType.DMA,
        pltpu.SemaphoreType.DMA,
    ],
    compiler_params=pltpu.CompilerParams(needs_layout_passes=False),
)
def _triu_fused(
    x2d, desc_hbm, cmap_hbm, out_hbm,
    desc_v, cmap_v, slab0, slab1, outst0, outst1, insem, outsem,
):
    w = lax.axis_index("s") * 2 + lax.axis_index("c")
    bid0 = w * _PER_W
    j0 = bid0 // _NBB
    pltpu.sync_copy(
        desc_hbm.at[pl.ds(pl.multiple_of(j0 * _DSTRIDE, 8), 10 * _DSTRIDE)],
        desc_v,
    )
    pltpu.sync_copy(
        cmap_hbm.at[pl.ds(pl.multiple_of(j0 * 128, 128), 10 * 128)], cmap_v
    )

    def issue(bid, slab):
        j = bid // _NBB
        b0 = (bid % _NBB) * 128
        dbase = (j - j0) * _DSTRIDE
        d0 = desc_v[pl.ds(dbase, 16)]
        d1 = desc_v[pl.ds(dbase + 16, 16)]
        nr = d0[0]
        frs = [d0[1 + p] for p in range(15)] + [d1[0]]
        for p in range(_MAXP):
            @pl.when(p < nr)
            def _():
                pltpu.async_copy(
                    x2d.at[
                        pl.ds(pl.multiple_of(frs[p], 8), 16),
                        pl.ds(pl.multiple_of(b0, 128), 128),
                    ],
                    slab.at[pl.ds(p * 16, 16), :],
                    insem,
                )

    def drain_in(bid):
        nr = desc_v[pl.ds((bid // _NBB - j0) * _DSTRIDE, 16)][0]

        def wbody(p, c):
            pltpu.make_async_copy(
                x2d.at[pl.ds(0, 16), pl.ds(0, 128)],
                slab0.at[pl.ds(0, 16), :],
                insem,
            ).wait()
            return c

        lax.fori_loop(0, nr, wbody, 0)

    def compute(bid, slab, outst):
        j = bid // _NBB
        b0 = (bid % _NBB) * 128
        cmb = (j - j0) * 128
        cms = [cmap_v[pl.ds(cmb + 16 * t, 16)] for t in range(8)]

        def bbody(b, c):
            sp = jnp.full((16,), b, jnp.int32)
            for t in range(8):
                g = plsc.load_gather(slab, [cms[t], sp])
                outst[b, pl.ds(16 * t, 16)] = g
            return c

        lax.fori_loop(0, 128, bbody, 0)
        pltpu.async_copy(
            outst,
            out_hbm.at[
                pl.ds(pl.multiple_of(b0, 128), 128),
                pl.ds(pl.multiple_of(j * 128, 128), 128),
            ],
            outsem,
        )

    def drain_out():
        pltpu.make_async_copy(
            outst0, out_hbm.at[pl.ds(0, 128), pl.ds(0, 128)], outsem
        ).wait()

    issue(bid0, slab0)
    issue(bid0 + 1, slab1)

    def outer(t, carry):
        q0 = bid0 + 2 * t
        drain_in(q0)

        @pl.when(t > 0)
        def _():
            drain_out()

        compute(q0, slab0, outst0)

        @pl.when(2 * t + 2 < _PER_W)
        def _():
            issue(q0 + 2, slab0)

        drain_in(q0 + 1)

        @pl.when(t > 0)
        def _():
            drain_out()

        compute(q0 + 1, slab1, outst1)

        @pl.when(2 * t + 3 < _PER_W)
        def _():
            issue(q0 + 3, slab1)

        return carry

    lax.fori_loop(0, (_PER_W - 1) // 2, outer, 0)

    q = bid0 + _PER_W - 1
    drain_in(q)
    drain_out()
    compute(q, slab0, outst0)
    drain_out()
    drain_out()


def kernel(input):
    xt = jnp.transpose(input, (1, 2, 3, 0)).reshape(_H * _N * _N, _B)
    return _triu_fused(xt, jnp.asarray(_DESC), jnp.asarray(_CMAP))


# speedup vs baseline: 2.3867x; 1.8368x over previous
import functools

import jax
import jax.numpy as jnp
import numpy as np
from jax import lax
from jax.experimental import pallas as pl
from jax.experimental.pallas import tpu as pltpu
from jax.experimental.pallas import tpu_sc as plsc

_N = 64
_TRI = _N * (_N + 1) // 2
_H = 16
_B = 1024
_NCOLS = _H * _TRI
_NJ = _NCOLS // 128
_NBB = _B // 128
_NBLK = _NJ * _NBB
_PER_W = _NBLK // 32
_MAXP = 16
_DSTRIDE = 24
_TROWS = 272

_r_, _c_ = np.triu_indices(_N)
_OFFS = np.concatenate([[0], np.cumsum(np.arange(_N, 0, -1))])
_cols = np.arange(_NCOLS)
_h = _cols // _TRI
_k = _cols % _TRI
_rm = np.searchsorted(_OFFS, _k, side="right") - 1
_cm = _k - _OFFS[_rm] + _rm
_FLAT = _h * (_N * _N) + _rm * _N + _cm

_DESC = np.zeros((_TROWS, _DSTRIDE), np.int32)
_CMAP = np.zeros((_TROWS, 128), np.int32)
for _j in range(_NJ):
    _f = _FLAT[_j * 128 : (_j + 1) * 128]
    _runs = []
    _s = _p = int(_f[0])
    for _v in _f[1:]:
        _v = int(_v)
        if _v != _p + 1:
            _runs.append((_s, _p))
            _s = _v
        _p = _v
    _runs.append((_s, _p))
    _pieces = []
    for _a, _b2 in _runs:
        _ca = _a - (_a % 8)
        while _ca <= _b2:
            _pieces.append(min(_ca, _N * _N * _H - 16))
            _ca += 16
    assert len(_pieces) <= _MAXP
    _DESC[_j, 0] = len(_pieces)
    _DESC[_j, 1 : 1 + len(_pieces)] = _pieces
    for _q, _fl in enumerate(_f):
        for _pi, _ps in enumerate(_pieces):
            if _ps <= _fl < _ps + 16:
                _CMAP[_j, _q] = _pi * 16 + (int(_fl) - _ps)
                break
        else:
            raise AssertionError("uncovered column")
_DESC = _DESC.reshape(-1)
_CMAP = _CMAP.reshape(-1)

_mesh = plsc.VectorSubcoreMesh(core_axis_name="c", subcore_axis_name="s")


@functools.partial(
    pl.kernel,
    out_type=jax.ShapeDtypeStruct((_B, _NCOLS), jnp.float32),
    mesh=_mesh,
    scratch_types=[
        pltpu.VMEM((10 * _DSTRIDE,), jnp.int32),
        pltpu.VMEM((10 * 128,), jnp.int32),
        pltpu.VMEM((_MAXP * 16, 128), jnp.float32),
        pltpu.VMEM((_MAXP * 16, 128), jnp.float32),
        pltpu.VMEM((128, 128), jnp.float32),
        pltpu.VMEM((128, 128), jnp.float32),
        pltpu.SemaphoreType.DMA,
        pltpu.SemaphoreType.DMA,
    ],
    compiler_params=pltpu.CompilerParams(needs_layout_passes=False),
)
def _triu_fused(
    x2d, desc_hbm, cmap_hbm, out_hbm,
    desc_v, cmap_v, slab0, slab1, outst0, outst1, insem, outsem,
):
    w = lax.axis_index("s") * 2 + lax.axis_index("c")
    bid0 = w * _PER_W
    j0 = bid0 // _NBB
    pltpu.sync_copy(
        desc_hbm.at[pl.ds(pl.multiple_of(j0 * _DSTRIDE, 8), 10 * _DSTRIDE)],
        desc_v,
    )
    pltpu.sync_copy(
        cmap_hbm.at[pl.ds(pl.multiple_of(j0 * 128, 128), 10 * 128)], cmap_v
    )

    def issue(bid, slab):
        j = bid // _NBB
        b0 = (bid % _NBB) * 128
        dbase = (j - j0) * _DSTRIDE
        d0 = desc_v[pl.ds(dbase, 16)]
        d1 = desc_v[pl.ds(dbase + 16, 16)]
        nr = d0[0]
        frs = [d0[1 + p] for p in range(15)] + [d1[0]]
        for p in range(_MAXP):
            @pl.when(p < nr)
            def _():
                pltpu.async_copy(
                    x2d.at[
                        pl.ds(pl.multiple_of(frs[p], 8), 16),
                        pl.ds(pl.multiple_of(b0, 128), 128),
                    ],
                    slab.at[pl.ds(p * 16, 16), :],
                    insem,
                )

    def drain_in(bid):
        nr = desc_v[pl.ds((bid // _NBB - j0) * _DSTRIDE, 16)][0]

        def wbody(p, c):
            pltpu.make_async_copy(
                x2d.at[pl.ds(0, 16), pl.ds(0, 128)],
                slab0.at[pl.ds(0, 16), :],
                insem,
            ).wait()
            return c

        lax.fori_loop(0, nr, wbody, 0)

    def compute(bid, slab, outst):
        j = bid // _NBB
        b0 = (bid % _NBB) * 128
        cmb = (j - j0) * 128
        cms = [cmap_v[pl.ds(cmb + 16 * t, 16)] for t in range(8)]

        def bbody(b, colv):
            for t in range(8):
                g = plsc.load_gather(slab, [cms[t], colv])
                outst[b, pl.ds(16 * t, 16)] = g
            return colv + 1

        plsc.parallel_loop(
            0, 128, 1, unroll=4, carry=jnp.zeros((16,), jnp.int32)
        )(bbody)
        pltpu.async_copy(
            outst,
            out_hbm.at[
                pl.ds(pl.multiple_of(b0, 128), 128),
                pl.ds(pl.multiple_of(j * 128, 128), 128),
            ],
            outsem,
        )

    def drain_out():
        pltpu.make_async_copy(
            outst0, out_hbm.at[pl.ds(0, 128), pl.ds(0, 128)], outsem
        ).wait()

    issue(bid0, slab0)
    issue(bid0 + 1, slab1)

    def outer(t, carry):
        q0 = bid0 + 2 * t
        drain_in(q0)

        @pl.when(t > 0)
        def _():
            drain_out()

        compute(q0, slab0, outst0)

        @pl.when(2 * t + 2 < _PER_W)
        def _():
            issue(q0 + 2, slab0)

        drain_in(q0 + 1)

        @pl.when(t > 0)
        def _():
            drain_out()

        compute(q0 + 1, slab1, outst1)

        @pl.when(2 * t + 3 < _PER_W)
        def _():
            issue(q0 + 3, slab1)

        return carry

    lax.fori_loop(0, (_PER_W - 1) // 2, outer, 0)

    q = bid0 + _PER_W - 1
    drain_in(q)
    drain_out()
    compute(q, slab0, outst0)
    drain_out()
    drain_out()


def kernel(input):
    xt = jnp.transpose(input, (1, 2, 3, 0)).reshape(_H * _N * _N, _B)
    return _triu_fused(xt, jnp.asarray(_DESC), jnp.asarray(_CMAP))


# speedup vs baseline: 8.0446x; 3.3706x over previous
import functools

import jax
import jax.numpy as jnp
import numpy as np
from jax import lax
from jax.experimental import pallas as pl
from jax.experimental.pallas import tpu as pltpu
from jax.experimental.pallas import tpu_sc as plsc

_N = 64
_TRI = _N * (_N + 1) // 2
_H = 16
_B = 1024
_NCOLS = _H * _TRI
_NJ = _NCOLS // 128
_NBB = _B // 128
_NBLK = _NJ * _NBB
_PER_W = _NBLK // 32
_MAXP = 16
_DSTRIDE = 24
_TROWS = 272

_r_, _c_ = np.triu_indices(_N)
_OFFS = np.concatenate([[0], np.cumsum(np.arange(_N, 0, -1))])
_cols = np.arange(_NCOLS)
_h = _cols // _TRI
_k = _cols % _TRI
_rm = np.searchsorted(_OFFS, _k, side="right") - 1
_cm = _k - _OFFS[_rm] + _rm
_FLAT = _h * (_N * _N) + _rm * _N + _cm

_DESC = np.zeros((_TROWS, _DSTRIDE), np.int32)
_CMAP = np.zeros((_TROWS, 128), np.int32)
for _j in range(_NJ):
    _f = _FLAT[_j * 128 : (_j + 1) * 128]
    _runs = []
    _s = _p = int(_f[0])
    for _v in _f[1:]:
        _v = int(_v)
        if _v != _p + 1:
            _runs.append((_s, _p))
            _s = _v
        _p = _v
    _runs.append((_s, _p))
    _pieces = []
    for _a, _b2 in _runs:
        _ca = _a - (_a % 8)
        while _ca <= _b2:
            _pieces.append(min(_ca, _N * _N * _H - 16))
            _ca += 16
    assert len(_pieces) <= _MAXP
    _DESC[_j, 0] = len(_pieces)
    _DESC[_j, 1 : 1 + len(_pieces)] = _pieces
    for _q, _fl in enumerate(_f):
        for _pi, _ps in enumerate(_pieces):
            if _ps <= _fl < _ps + 16:
                _CMAP[_j, _q] = _pi * 16 + (int(_fl) - _ps)
                break
        else:
            raise AssertionError("uncovered column")
_DESC = _DESC.reshape(-1)
_CMAP = _CMAP.reshape(-1)

_mesh = plsc.VectorSubcoreMesh(core_axis_name="c", subcore_axis_name="s")


@functools.partial(
    pl.kernel,
    out_type=jax.ShapeDtypeStruct((_B, _NCOLS), jnp.float32),
    mesh=_mesh,
    scratch_types=[
        pltpu.VMEM((10 * _DSTRIDE,), jnp.int32),
        pltpu.VMEM((10 * 128,), jnp.int32),
        pltpu.VMEM((_MAXP * 16, 128), jnp.float32),
        pltpu.VMEM((_MAXP * 16, 128), jnp.float32),
        pltpu.VMEM((128, 128), jnp.float32),
        pltpu.VMEM((128, 128), jnp.float32),
        pltpu.SemaphoreType.DMA,
        pltpu.SemaphoreType.DMA,
    ],
    compiler_params=pltpu.CompilerParams(needs_layout_passes=False),
)
def _triu_fused(
    x2d, desc_hbm, cmap_hbm, out_hbm,
    desc_v, cmap_v, slab0, slab1, outst0, outst1, insem, outsem,
):
    w = lax.axis_index("s") * 2 + lax.axis_index("c")
    bid0 = w * _PER_W
    j0 = bid0 // _NBB
    pltpu.sync_copy(
        desc_hbm.at[pl.ds(pl.multiple_of(j0 * _DSTRIDE, 8), 10 * _DSTRIDE)],
        desc_v,
    )
    pltpu.sync_copy(
        cmap_hbm.at[pl.ds(pl.multiple_of(j0 * 128, 128), 10 * 128)], cmap_v
    )

    def issue(bid, slab):
        j = bid // _NBB
        b0 = (bid % _NBB) * 128
        dbase = (j - j0) * _DSTRIDE
        d0 = desc_v[pl.ds(dbase, 16)]
        d1 = desc_v[pl.ds(dbase + 16, 16)]
        nr = d0[0]
        frs = [d0[1 + p] for p in range(15)] + [d1[0]]
        for p in range(_MAXP):
            @pl.when(p < nr)
            def _():
                pltpu.async_copy(
                    x2d.at[
                        pl.ds(pl.multiple_of(frs[p], 8), 16),
                        pl.ds(pl.multiple_of(b0, 128), 128),
                    ],
                    slab.at[pl.ds(p * 16, 16), :],
                    insem,
                )

    def drain_in(bid):
        nr = desc_v[pl.ds((bid // _NBB - j0) * _DSTRIDE, 16)][0]

        def wbody(p, c):
            pltpu.make_async_copy(
                x2d.at[pl.ds(0, 16), pl.ds(0, 128)],
                slab0.at[pl.ds(0, 16), :],
                insem,
            ).wait()
            return c

        lax.fori_loop(0, nr, wbody, 0)

    def compute(bid, slab, outst):
        j = bid // _NBB
        b0 = (bid % _NBB) * 128
        cmb = (j - j0) * 128
        iota = lax.broadcasted_iota(jnp.int32, (16,), 0)

        def tbody(tile):
            tb = (tile // 8) * 16
            tc = (tile % 8) * 16
            cmv = cmap_v[pl.ds(cmb + tc, 16)]
            civ = iota + tc
            tbv = jnp.full((16,), tb, jnp.int32)
            rotv = iota
            for d in range(16):
                bv = tbv + rotv
                g = plsc.load_gather(slab, [cmv, bv])
                plsc.store_scatter(outst, [bv, civ], g)
                rotv = (rotv + 1) & 15

        plsc.parallel_loop(0, 64, 1, unroll=2)(tbody)
        pltpu.async_copy(
            outst,
            out_hbm.at[
                pl.ds(pl.multiple_of(b0, 128), 128),
                pl.ds(pl.multiple_of(j * 128, 128), 128),
            ],
            outsem,
        )

    def drain_out():
        pltpu.make_async_copy(
            outst0, out_hbm.at[pl.ds(0, 128), pl.ds(0, 128)], outsem
        ).wait()

    issue(bid0, slab0)
    issue(bid0 + 1, slab1)

    def outer(t, carry):
        q0 = bid0 + 2 * t
        drain_in(q0)

        @pl.when(t > 0)
        def _():
            drain_out()

        compute(q0, slab0, outst0)

        @pl.when(2 * t + 2 < _PER_W)
        def _():
            issue(q0 + 2, slab0)

        drain_in(q0 + 1)

        @pl.when(t > 0)
        def _():
            drain_out()

        compute(q0 + 1, slab1, outst1)

        @pl.when(2 * t + 3 < _PER_W)
        def _():
            issue(q0 + 3, slab1)

        return carry

    lax.fori_loop(0, (_PER_W - 1) // 2, outer, 0)

    q = bid0 + _PER_W - 1
    drain_in(q)
    drain_out()
    compute(q, slab0, outst0)
    drain_out()
    drain_out()


def kernel(input):
    xt = jnp.transpose(input, (1, 2, 3, 0)).reshape(_H * _N * _N, _B)
    return _triu_fused(xt, jnp.asarray(_DESC), jnp.asarray(_CMAP))
